# Initial kernel scaffold; baseline (speedup 1.0000x reference)
import jax
import jax.numpy as jnp
from jax.experimental import pallas as pl


def kernel(x, pos, edge_index, batch, num_graphs, emb_table, emb_W, emb_b, W_in, filt_W, filt_b, W_out, b_out, fc_W1, fc_b1, fc_W2, fc_b2, head_W1, head_b1, head_W2, head_b2):
    raise NotImplementedError("write your pallas kernel here")


# speedup vs baseline: 2.6072x; 2.6072x over previous
import functools

import numpy as np
import jax
import jax.numpy as jnp
from jax import lax
from jax.experimental import pallas as pl
from jax.experimental.pallas import tpu as pltpu
from jax.experimental.pallas import tpu_sc as plsc

N = 10000
E = 160000
D = 256
MD = 128
NR = 32
NB = 4
G = 32
NT = 101
OUT = 10
CUTOFF = 5.0
P = 6

BN = 1000
NSTEP = N // BN
BE = 2000
ESTEP = E // BE
CE = 128
NCHUNK = E // CE
NW = 32
WCH = (NCHUNK + NW - 1) // NW
NPAD = 10240
RPT = NPAD // 16

_sc_mesh = plsc.VectorSubcoreMesh(core_axis_name="c", subcore_axis_name="s")


@functools.partial(
    pl.kernel,
    mesh=_sc_mesh,
    out_type=jax.ShapeDtypeStruct((E,), jnp.float32),
    scratch_types=[
        pltpu.VMEM((3 * N,), jnp.float32),
        pltpu.VMEM((CE,), jnp.int32),
        pltpu.VMEM((CE,), jnp.int32),
        pltpu.VMEM((CE,), jnp.float32),
    ],
    compiler_params=pltpu.CompilerParams(needs_layout_passes=False),
)
def _dist_kernel(pos_hbm, src_hbm, dst_hbm, d2_hbm, pos_v, srcb, dstb, outb):
    cid = lax.axis_index("c")
    sid = lax.axis_index("s")
    w = sid * 2 + cid
    pltpu.sync_copy(pos_hbm, pos_v)

    def chunk(i, carry):
        c = w + NW * i

        @pl.when(c < NCHUNK)
        def _():
            base = c * CE
            pltpu.sync_copy(src_hbm.at[pl.ds(base, CE)], srcb)
            pltpu.sync_copy(dst_hbm.at[pl.ds(base, CE)], dstb)
            for g in range(CE // 16):
                si = srcb[pl.ds(g * 16, 16)] * 3
                di = dstb[pl.ds(g * 16, 16)] * 3
                dx = plsc.load_gather(pos_v, [si]) - plsc.load_gather(pos_v, [di])
                dy = plsc.load_gather(pos_v, [si + 1]) - plsc.load_gather(pos_v, [di + 1])
                dz = plsc.load_gather(pos_v, [si + 2]) - plsc.load_gather(pos_v, [di + 2])
                outb[pl.ds(g * 16, 16)] = dx * dx + dy * dy + dz * dz
            pltpu.sync_copy(outb, d2_hbm.at[pl.ds(base, CE)])

        return carry

    lax.fori_loop(0, WCH, chunk, 0)


@functools.partial(
    pl.kernel,
    mesh=_sc_mesh,
    out_type=jax.ShapeDtypeStruct((2, NPAD, MD), jnp.float32),
    scratch_types=[
        pltpu.VMEM_SHARED((NPAD, MD), jnp.float32),
        pltpu.VMEM((CE,), jnp.int32),
        pltpu.VMEM((CE,), jnp.int32),
        pltpu.VMEM((CE, MD), jnp.float32),
        pltpu.VMEM((CE, MD), jnp.float32),
        pltpu.SemaphoreType.DMA,
    ],
    compiler_params=pltpu.CompilerParams(needs_layout_passes=False),
)
def _conv_kernel(hw_hbm, filt_hbm, src_hbm, dst_hbm, zeros_hbm, out_hbm,
                 acc, srcb, dstb, gb, fb, sem):
    cid = lax.axis_index("c")
    sid = lax.axis_index("s")
    w = sid * 2 + cid
    pltpu.sync_copy(zeros_hbm.at[pl.ds(sid * RPT, RPT)],
                    acc.at[pl.ds(sid * RPT, RPT)])
    plsc.subcore_barrier()

    def chunk(i, carry):
        c = w + NW * i

        @pl.when(c < NCHUNK)
        def _():
            base = c * CE
            pltpu.sync_copy(dst_hbm.at[pl.ds(base, CE)], dstb)
            pltpu.sync_copy(src_hbm.at[pl.ds(base, CE)], srcb)
            pltpu.async_copy(hw_hbm.at[dstb], gb, sem).wait()
            pltpu.sync_copy(filt_hbm.at[pl.ds(base, CE)], fb)

            def mul_row(e, cr):
                for j in range(MD // 16):
                    sl = pl.ds(j * 16, 16)
                    gb[e, sl] = gb[e, sl] * fb[e, sl]
                return cr

            lax.fori_loop(0, CE, mul_row, 0)
            pltpu.sync_copy(gb, acc.at[srcb], add=True)

        return carry

    lax.fori_loop(0, WCH, chunk, 0)
    plsc.subcore_barrier()
    pltpu.sync_copy(acc.at[pl.ds(sid * RPT, RPT)],
                    out_hbm.at[cid, pl.ds(sid * RPT, RPT)])


def _embed_body(x_ref, tab_ref, embw_ref, embb_ref, win0_ref, h_ref, hw_ref):
    xi = x_ref[...]
    fused = jnp.dot(tab_ref[...], embw_ref[...],
                    preferred_element_type=jnp.float32)
    oh = (lax.broadcasted_iota(jnp.int32, (BN, NT), 1) == xi).astype(jnp.float32)
    h0 = jax.nn.gelu(jnp.dot(oh, fused, preferred_element_type=jnp.float32)
                     + embb_ref[...])
    h_ref[...] = h0
    hw_ref[...] = jnp.dot(h0, win0_ref[...], preferred_element_type=jnp.float32)


_embed_call = pl.pallas_call(
    _embed_body,
    grid=(NSTEP,),
    in_specs=[
        pl.BlockSpec((BN, 1), lambda i: (i, 0)),
        pl.BlockSpec((NT, 5), lambda i: (0, 0)),
        pl.BlockSpec((5, D), lambda i: (0, 0)),
        pl.BlockSpec((1, D), lambda i: (0, 0)),
        pl.BlockSpec((D, MD), lambda i: (0, 0)),
    ],
    out_specs=[
        pl.BlockSpec((BN, D), lambda i: (i, 0)),
        pl.BlockSpec((BN, MD), lambda i: (i, 0)),
    ],
    out_shape=[
        jax.ShapeDtypeStruct((N, D), jnp.float32),
        jax.ShapeDtypeStruct((N, MD), jnp.float32),
    ],
)


def _filt_body(d2_ref, fw_ref, fb_ref, out_ref):
    d2 = d2_ref[...]
    dist = jnp.sqrt(d2 + 1e-12)
    d = jnp.maximum(dist / CUTOFF, 1e-6)
    dsq = d * d
    d4 = dsq * dsq
    d5 = d4 * d
    a = -(P + 1) * (P + 2) / 2.0
    b = P * (P + 2)
    c = -P * (P + 1) / 2.0
    env = 1.0 / d + a * d5 + b * d5 * d + c * d5 * dsq
    env = jnp.where(dist < CUTOFF, env, 0.0) * np.float32(np.sqrt(2.0 / CUTOFF))
    freq = (lax.broadcasted_iota(jnp.int32, (1, NR), 1).astype(jnp.float32)
            + 1.0) * np.float32(np.pi)
    rbf = env * jnp.sin(d * freq)
    for blk in range(NB):
        out_ref[blk] = (jnp.dot(rbf, fw_ref[blk], preferred_element_type=jnp.float32)
                        + fb_ref[blk])


_filt_call = pl.pallas_call(
    _filt_body,
    grid=(ESTEP,),
    in_specs=[
        pl.BlockSpec((BE, 1), lambda i: (i, 0)),
        pl.BlockSpec((NB, NR, MD), lambda i: (0, 0, 0)),
        pl.BlockSpec((NB, 1, MD), lambda i: (0, 0, 0)),
    ],
    out_specs=pl.BlockSpec((NB, BE, MD), lambda i: (0, i, 0)),
    out_shape=jax.ShapeDtypeStruct((NB, E, MD), jnp.float32),
)


def _block_body(parts_ref, h_ref, wout_ref, bout_ref, w1_ref, b1_ref,
                w2_ref, b2_ref, winn_ref, h_out_ref, hw_out_ref):
    agg = parts_ref[0] + parts_ref[1]
    t = jax.nn.gelu(jnp.dot(agg, wout_ref[...],
                            preferred_element_type=jnp.float32) + bout_ref[...])
    h1 = h_ref[...] + t
    u = jax.nn.gelu(jnp.dot(h1, w1_ref[...],
                            preferred_element_type=jnp.float32) + b1_ref[...])
    h2 = h1 + jnp.dot(u, w2_ref[...], preferred_element_type=jnp.float32) + b2_ref[...]
    h_out_ref[...] = h2
    hw_out_ref[...] = jnp.dot(h2, winn_ref[...], preferred_element_type=jnp.float32)


_block_call = pl.pallas_call(
    _block_body,
    grid=(NSTEP,),
    in_specs=[
        pl.BlockSpec((2, BN, MD), lambda i: (0, i, 0)),
        pl.BlockSpec((BN, D), lambda i: (i, 0)),
        pl.BlockSpec((MD, D), lambda i: (0, 0)),
        pl.BlockSpec((1, D), lambda i: (0, 0)),
        pl.BlockSpec((D, D), lambda i: (0, 0)),
        pl.BlockSpec((1, D), lambda i: (0, 0)),
        pl.BlockSpec((D, D), lambda i: (0, 0)),
        pl.BlockSpec((1, D), lambda i: (0, 0)),
        pl.BlockSpec((D, MD), lambda i: (0, 0)),
    ],
    out_specs=[
        pl.BlockSpec((BN, D), lambda i: (i, 0)),
        pl.BlockSpec((BN, MD), lambda i: (i, 0)),
    ],
    out_shape=[
        jax.ShapeDtypeStruct((N, D), jnp.float32),
        jax.ShapeDtypeStruct((N, MD), jnp.float32),
    ],
)


def _pool_body(b3_ref, h_ref, hw1_ref, hb1_ref, hw2_ref, hb2_ref, out_ref,
               gsum, cnt):
    i = pl.program_id(0)

    @pl.when(i == 0)
    def _():
        gsum[...] = jnp.zeros_like(gsum)
        cnt[...] = jnp.zeros_like(cnt)

    bb = jnp.minimum(b3_ref[0], G - 1)
    oh = (lax.broadcasted_iota(jnp.int32, (G, BN), 0) == bb).astype(jnp.float32)
    gsum[...] += jnp.dot(oh, h_ref[...], preferred_element_type=jnp.float32)
    cnt[...] += jnp.sum(oh, axis=1, keepdims=True)

    @pl.when(i == NSTEP - 1)
    def _():
        g = gsum[...] / jnp.maximum(cnt[...], 1.0)
        z = jax.nn.gelu(jnp.dot(g, hw1_ref[...],
                                preferred_element_type=jnp.float32) + hb1_ref[...])
        out_ref[...] = jnp.dot(z, hw2_ref[...],
                               preferred_element_type=jnp.float32) + hb2_ref[...]


_pool_call = pl.pallas_call(
    _pool_body,
    grid=(NSTEP,),
    in_specs=[
        pl.BlockSpec((1, 1, BN), lambda i: (i, 0, 0)),
        pl.BlockSpec((BN, D), lambda i: (i, 0)),
        pl.BlockSpec((D, D), lambda i: (0, 0)),
        pl.BlockSpec((1, D), lambda i: (0, 0)),
        pl.BlockSpec((D, OUT), lambda i: (0, 0)),
        pl.BlockSpec((1, OUT), lambda i: (0, 0)),
    ],
    out_specs=pl.BlockSpec((G, OUT), lambda i: (0, 0)),
    out_shape=jax.ShapeDtypeStruct((G, OUT), jnp.float32),
    scratch_shapes=[
        pltpu.VMEM((G, D), jnp.float32),
        pltpu.VMEM((G, 1), jnp.float32),
    ],
)


def kernel(x, pos, edge_index, batch, num_graphs, emb_table, emb_W, emb_b,
           W_in, filt_W, filt_b, W_out, b_out, fc_W1, fc_b1, fc_W2, fc_b2,
           head_W1, head_b1, head_W2, head_b2):
    del num_graphs
    pos_flat = pos.reshape(-1)
    src = edge_index[0]
    dst = edge_index[1]

    d2 = _dist_kernel(pos_flat, src, dst)
    filt_all = _filt_call(d2.reshape(E, 1), filt_W, filt_b.reshape(NB, 1, MD))
    h, hw = _embed_call(x, emb_table, emb_W, emb_b.reshape(1, D), W_in[0])

    zeros = jnp.zeros((NPAD, MD), jnp.float32)
    for blk in range(NB):
        parts = _conv_kernel(hw, filt_all[blk], src, dst, zeros)
        h, hw = _block_call(parts, h,
                            W_out[blk], b_out[blk].reshape(1, D),
                            fc_W1[blk], fc_b1[blk].reshape(1, D),
                            fc_W2[blk], fc_b2[blk].reshape(1, D),
                            W_in[(blk + 1) % NB])

    return _pool_call(batch.reshape(NSTEP, 1, BN), h,
                      head_W1, head_b1.reshape(1, D),
                      head_W2, head_b2.reshape(1, OUT))
